# Initial kernel scaffold; baseline (speedup 1.0000x reference)
import jax
import jax.numpy as jnp
from jax.experimental import pallas as pl


def kernel(src, table, W, b):
    raise NotImplementedError("write your pallas kernel here")


# speedup vs baseline: 3.2889x; 3.2889x over previous
import functools

import jax
import jax.numpy as jnp
from jax import lax
from jax.experimental import pallas as pl
from jax.experimental.pallas import tpu as pltpu
from jax.experimental.pallas import tpu_sc as plsc

B = 4096
LSEQ = 50
EMB = 256
OUT = 1000
VOCAB = 100000

NC = 2
NS = 16
LANES = 16
NW = NC * NS
BPW = B // NW
NREG = EMB // LANES


def _sc_pool_body(src_hbm, table_hbm, out_hbm, idx_v, rows0, rows1, acc_v,
                  sem0, sem1):
    c = lax.axis_index("c")
    s = lax.axis_index("s")
    wid = s * NC + c
    base = wid * BPW

    pltpu.sync_copy(src_hbm.at[pl.ds(base, BPW), :], idx_v)

    pltpu.async_copy(table_hbm.at[idx_v.at[0]], rows0, sem0)
    pltpu.async_copy(table_hbm.at[idx_v.at[1]], rows1, sem1)

    def process(rows, row_i):
        def jbody(j, accs):
            return tuple(accs[k] + rows[j, pl.ds(k * LANES, LANES)]
                         for k in range(NREG))
        zero = jnp.zeros((LANES,), jnp.float32)
        accs = lax.fori_loop(0, LSEQ, jbody, (zero,) * NREG)
        for k in range(NREG):
            acc_v[row_i, pl.ds(k * LANES, LANES)] = accs[k]

    def obody(i, carry):
        r0 = 2 * i
        pltpu.make_async_copy(table_hbm.at[idx_v.at[r0]], rows0, sem0).wait()
        process(rows0, r0)

        @pl.when(r0 + 2 < BPW)
        def _():
            pltpu.async_copy(table_hbm.at[idx_v.at[r0 + 2]], rows0, sem0)

        pltpu.make_async_copy(table_hbm.at[idx_v.at[r0 + 1]], rows1,
                              sem1).wait()
        process(rows1, r0 + 1)

        @pl.when(r0 + 3 < BPW)
        def _():
            pltpu.async_copy(table_hbm.at[idx_v.at[r0 + 3]], rows1, sem1)

        return carry

    lax.fori_loop(0, BPW // 2, obody, 0)

    pltpu.sync_copy(acc_v, out_hbm.at[pl.ds(base, BPW), :])


@functools.partial(jax.jit, static_argnames=())
def _sc_pool(src, table):
    mesh = plsc.VectorSubcoreMesh(core_axis_name="c", subcore_axis_name="s")
    f = pl.kernel(
        _sc_pool_body,
        out_type=jax.ShapeDtypeStruct((B, EMB), jnp.float32),
        mesh=mesh,
        scratch_types=[
            pltpu.VMEM((BPW, LSEQ), jnp.int32),
            pltpu.VMEM((LSEQ, EMB), jnp.float32),
            pltpu.VMEM((LSEQ, EMB), jnp.float32),
            pltpu.VMEM((BPW, EMB), jnp.float32),
            pltpu.SemaphoreType.DMA,
            pltpu.SemaphoreType.DMA,
        ],
    )
    return f(src, table)


def _tc_finish_body(emb_ref, src_ref, w_ref, b_ref, out_ref):
    x = emb_ref[...]
    cnt = jnp.sum((src_ref[...] != 0).astype(jnp.float32), axis=1,
                  keepdims=True)
    x = x / cnt
    logits = lax.dot_general(x, w_ref[...], (((1,), (1,)), ((), ())),
                             preferred_element_type=jnp.float32,
                             precision=lax.Precision.HIGHEST)
    logits = logits + b_ref[...]
    m = jnp.max(logits, axis=-1, keepdims=True)
    sh = logits - m
    lse = jnp.log(jnp.sum(jnp.exp(sh), axis=-1, keepdims=True))
    out_ref[...] = sh - lse


def _tc_finish(emb_sum, src, W, b):
    BB = 512
    return pl.pallas_call(
        _tc_finish_body,
        grid=(B // BB,),
        in_specs=[
            pl.BlockSpec((BB, EMB), lambda i: (i, 0)),
            pl.BlockSpec((BB, LSEQ), lambda i: (i, 0)),
            pl.BlockSpec((OUT, EMB), lambda i: (0, 0)),
            pl.BlockSpec((1, OUT), lambda i: (0, 0)),
        ],
        out_specs=pl.BlockSpec((BB, OUT), lambda i: (i, 0)),
        out_shape=jax.ShapeDtypeStruct((B, OUT), jnp.float32),
    )(emb_sum, src, W, b.reshape(1, OUT))


def kernel(src, table, W, b):
    emb_sum = _sc_pool(src, table)
    return _tc_finish(emb_sum, src, W, b)
